# Initial kernel scaffold; baseline (speedup 1.0000x reference)
import functools

import jax
import jax.numpy as jnp
from jax import lax
from jax.experimental import pallas as pl
from jax.experimental.pallas import tpu as pltpu
from jax.experimental.pallas import tpu_sc as plsc

H = 128
G = 16
NUM_CORES = 2
NUM_SUBCORES = 16
NUM_TILES = NUM_CORES * NUM_SUBCORES
ROWS_PER_TILE = 320
N_PAD = NUM_TILES * ROWS_PER_TILE
CH = 128
ROW_BLK = 512
HIGH = lax.Precision.HIGHEST


def _sc_agg_body(chunks_per_tile, h_hbm, src_hbm, dst_hbm, zero_hbm, out_hbm,
                 sidx, didx, rows, acc_sh, sem):
    cid = lax.axis_index("c")
    sid = lax.axis_index("s")
    wid = cid * NUM_SUBCORES + sid
    pltpu.sync_copy(zero_hbm.at[pl.ds(sid * ROWS_PER_TILE, ROWS_PER_TILE)],
                    acc_sh.at[pl.ds(sid * ROWS_PER_TILE, ROWS_PER_TILE)])
    plsc.subcore_barrier()

    edges_per_tile = CH * chunks_per_tile
    base = wid * edges_per_tile

    def body(c, carry):
        off = base + c * CH
        pltpu.sync_copy(src_hbm.at[pl.ds(off, CH)], sidx)
        pltpu.sync_copy(dst_hbm.at[pl.ds(off, CH)], didx)
        pltpu.async_copy(h_hbm.at[sidx], rows, sem).wait()
        pltpu.sync_copy(rows, acc_sh.at[didx], add=True)
        return carry

    lax.fori_loop(0, chunks_per_tile, body, 0)
    plsc.subcore_barrier()
    pltpu.sync_copy(acc_sh.at[pl.ds(sid * ROWS_PER_TILE, ROWS_PER_TILE)],
                    out_hbm.at[cid, pl.ds(sid * ROWS_PER_TILE, ROWS_PER_TILE)])


def _sc_agg(h, srcp, dstp, zeros, chunks_per_tile):
    mesh = plsc.VectorSubcoreMesh(core_axis_name="c", subcore_axis_name="s")
    return pl.kernel(
        functools.partial(_sc_agg_body, chunks_per_tile),
        out_type=jax.ShapeDtypeStruct((NUM_CORES, N_PAD, H), jnp.float32),
        mesh=mesh,
        scratch_types=[
            pltpu.VMEM((CH,), jnp.int32),
            pltpu.VMEM((CH,), jnp.int32),
            pltpu.VMEM((CH, H), jnp.float32),
            pltpu.VMEM_SHARED((N_PAD, H), jnp.float32),
            pltpu.SemaphoreType.DMA,
        ],
    )(h, srcp, dstp, zeros)


def _sc_deg_body(chunks_per_tile, dst_hbm, out_hbm, didx_all, hist):
    cid = lax.axis_index("c")
    sid = lax.axis_index("s")
    wid = cid * NUM_SUBCORES + sid
    edges_per_tile = CH * chunks_per_tile

    def zbody(i, carry):
        hist[pl.ds(i * 16, 16)] = jnp.zeros((16,), jnp.float32)
        return carry

    lax.fori_loop(0, N_PAD // 16, zbody, 0)
    pltpu.sync_copy(dst_hbm.at[pl.ds(wid * edges_per_tile, edges_per_tile)],
                    didx_all)

    ones16 = jnp.ones((16,), jnp.float32)

    def body(g, carry):
        idx = didx_all[pl.ds(g * 16, 16)]
        plsc.addupdate_scatter(hist, [idx], ones16)
        return carry

    lax.fori_loop(0, edges_per_tile // 16, body, 0)
    pltpu.sync_copy(hist, out_hbm.at[wid])


def _sc_deg(dstp, chunks_per_tile):
    mesh = plsc.VectorSubcoreMesh(core_axis_name="c", subcore_axis_name="s")
    edges_per_tile = CH * chunks_per_tile
    return pl.kernel(
        functools.partial(_sc_deg_body, chunks_per_tile),
        out_type=jax.ShapeDtypeStruct((NUM_TILES, N_PAD), jnp.float32),
        mesh=mesh,
        scratch_types=[
            pltpu.VMEM((edges_per_tile,), jnp.int32),
            pltpu.VMEM((N_PAD,), jnp.float32),
        ],
    )(dstp)


def _embed_body(x_ref, w_ref, b_ref, o_ref):
    o_ref[...] = jnp.maximum(
        jnp.dot(x_ref[...], w_ref[...], precision=HIGH) + b_ref[...], 0.0)


def _embed(x_pad, W_emb, b_emb):
    grid = (N_PAD // ROW_BLK,)
    return pl.pallas_call(
        _embed_body,
        grid=grid,
        in_specs=[
            pl.BlockSpec((ROW_BLK, H), lambda i: (i, 0)),
            pl.BlockSpec((H, H), lambda i: (0, 0)),
            pl.BlockSpec((1, H), lambda i: (0, 0)),
        ],
        out_specs=pl.BlockSpec((ROW_BLK, H), lambda i: (i, 0)),
        out_shape=jax.ShapeDtypeStruct((N_PAD, H), jnp.float32),
    )(x_pad, W_emb, b_emb.reshape(1, H))


def _recip_body(h_ref, o_ref):
    deg = jnp.sum(h_ref[...], axis=0)
    o_ref[...] = (1.0 / jnp.maximum(deg, 1.0))[:, None]


def _recip_deg(hists):
    return pl.pallas_call(
        _recip_body,
        out_shape=jax.ShapeDtypeStruct((N_PAD, 1), jnp.float32),
    )(hists)


def _layer_body(h_ref, a0_ref, a1_ref, rd_ref, ws_ref, wn_ref, b_ref, o_ref):
    agg = (a0_ref[...] + a1_ref[...]) * rd_ref[...]
    o_ref[...] = jnp.maximum(
        jnp.dot(h_ref[...], ws_ref[...], precision=HIGH)
        + jnp.dot(agg, wn_ref[...], precision=HIGH) + b_ref[...], 0.0)


def _layer(h, agg2, rd, Ws, Wn, b):
    grid = (N_PAD // ROW_BLK,)
    row_spec = pl.BlockSpec((ROW_BLK, H), lambda i: (i, 0))
    w_spec = pl.BlockSpec((H, H), lambda i: (0, 0))
    return pl.pallas_call(
        _layer_body,
        grid=grid,
        in_specs=[
            row_spec,
            row_spec,
            row_spec,
            pl.BlockSpec((ROW_BLK, 1), lambda i: (i, 0)),
            w_spec,
            w_spec,
            pl.BlockSpec((1, H), lambda i: (0, 0)),
        ],
        out_specs=row_spec,
        out_shape=jax.ShapeDtypeStruct((N_PAD, H), jnp.float32),
    )(h, agg2[0], agg2[1], rd, Ws, Wn, b.reshape(1, H))


def _pool_body(h_ref, bt_ref, w1_ref, b1_ref, w2_ref, b2_ref, o_ref,
               acc_ref, cnt_ref):
    i = pl.program_id(0)
    nsteps = pl.num_programs(0)

    @pl.when(i == 0)
    def _():
        acc_ref[...] = jnp.zeros_like(acc_ref)
        cnt_ref[...] = jnp.zeros_like(cnt_ref)

    b = bt_ref[0, 0, :]
    onehot = (b[:, None]
              == lax.broadcasted_iota(jnp.int32, (ROW_BLK, G), 1)
              ).astype(jnp.float32)
    acc_ref[...] += lax.dot_general(
        onehot, h_ref[...], (((0,), (0,)), ((), ())), precision=HIGH)
    cnt_ref[...] += lax.dot_general(
        onehot, jnp.ones((ROW_BLK, 1), jnp.float32),
        (((0,), (0,)), ((), ())), precision=HIGH)

    @pl.when(i == nsteps - 1)
    def _():
        pooled = acc_ref[...] / jnp.maximum(cnt_ref[...], 1.0)
        hid = jnp.maximum(
            jnp.dot(pooled, w1_ref[...], precision=HIGH) + b1_ref[...], 0.0)
        o_ref[...] = jnp.dot(hid, w2_ref[...], precision=HIGH) + b2_ref[...]


def _pool_head(h, batch3d, W1, b1, W2, b2):
    grid = (N_PAD // ROW_BLK,)
    return pl.pallas_call(
        _pool_body,
        grid=grid,
        in_specs=[
            pl.BlockSpec((ROW_BLK, H), lambda i: (i, 0)),
            pl.BlockSpec((1, 1, ROW_BLK), lambda i: (i, 0, 0)),
            pl.BlockSpec((H, H), lambda i: (0, 0)),
            pl.BlockSpec((1, H), lambda i: (0, 0)),
            pl.BlockSpec((H, 1), lambda i: (0, 0)),
            pl.BlockSpec((1, 1), lambda i: (0, 0)),
        ],
        out_specs=pl.BlockSpec((G, 1), lambda i: (0, 0)),
        out_shape=jax.ShapeDtypeStruct((G, 1), jnp.float32),
        scratch_shapes=[
            pltpu.VMEM((G, H), jnp.float32),
            pltpu.VMEM((G, 1), jnp.float32),
        ],
    )(h, batch3d, W1, b1.reshape(1, H), W2, b2.reshape(1, 1))


def kernel(x, edge_index, batch, W_emb, b_emb, W_self, W_nbr, b_conv,
           W1, b1, W2, b2):
    n = x.shape[0]
    e = edge_index.shape[1]
    num_layers = W_self.shape[0]

    chunks_per_tile = -(-e // (NUM_TILES * CH))
    e_pad = NUM_TILES * CH * chunks_per_tile

    src = edge_index[0]
    dst = edge_index[1]
    srcp = jnp.concatenate(
        [src, jnp.zeros((e_pad - e,), src.dtype)]).astype(jnp.int32)
    dstp = jnp.concatenate(
        [dst, jnp.full((e_pad - e,), N_PAD - 1, dst.dtype)]).astype(jnp.int32)

    x_pad = jnp.pad(x, ((0, N_PAD - n), (0, 0)))
    batch3d = jnp.pad(batch.astype(jnp.int32), (0, N_PAD - n),
                      constant_values=G).reshape(N_PAD // ROW_BLK, 1, ROW_BLK)
    zeros = jnp.zeros((N_PAD, H), jnp.float32)

    hists = _sc_deg(dstp, chunks_per_tile)
    rd = _recip_deg(hists)
    h = _embed(x_pad, W_emb, b_emb)
    for l in range(num_layers):
        agg2 = _sc_agg(h, srcp, dstp, zeros, chunks_per_tile)
        h = _layer(h, agg2, rd, W_self[l], W_nbr[l], b_conv[l])
    return _pool_head(h, batch3d, W1, b1, W2, b2)


# speedup vs baseline: 3.7251x; 3.7251x over previous
import functools

import jax
import jax.numpy as jnp
from jax import lax
from jax.experimental import pallas as pl
from jax.experimental.pallas import tpu as pltpu
from jax.experimental.pallas import tpu_sc as plsc

H = 128
G = 16
NUM_CORES = 2
NUM_SUBCORES = 16
NUM_TILES = NUM_CORES * NUM_SUBCORES
ROWS_PER_TILE = 320
N_PAD = NUM_TILES * ROWS_PER_TILE
CH = 128
ROW_BLK = 512
HIGH = lax.Precision.HIGHEST


def _sc_agg_body(chunks_per_tile, h_hbm, src_hbm, dst_hbm, zero_hbm, out_hbm,
                 sidx, didx, rows, acc_sh, sem):
    cid = lax.axis_index("c")
    sid = lax.axis_index("s")
    wid = cid * NUM_SUBCORES + sid
    pltpu.sync_copy(zero_hbm.at[pl.ds(sid * ROWS_PER_TILE, ROWS_PER_TILE)],
                    acc_sh.at[pl.ds(sid * ROWS_PER_TILE, ROWS_PER_TILE)])
    plsc.subcore_barrier()

    edges_per_tile = CH * chunks_per_tile
    base = wid * edges_per_tile

    def body(c, carry):
        off = base + c * CH
        pltpu.sync_copy(src_hbm.at[pl.ds(off, CH)], sidx)
        pltpu.sync_copy(dst_hbm.at[pl.ds(off, CH)], didx)
        pltpu.async_copy(h_hbm.at[sidx], rows, sem).wait()
        pltpu.sync_copy(rows, acc_sh.at[didx], add=True)
        return carry

    lax.fori_loop(0, chunks_per_tile, body, 0)
    plsc.subcore_barrier()
    pltpu.sync_copy(acc_sh.at[pl.ds(sid * ROWS_PER_TILE, ROWS_PER_TILE)],
                    out_hbm.at[cid, pl.ds(sid * ROWS_PER_TILE, ROWS_PER_TILE)])


def _sc_agg(h, srcp, dstp, zeros, chunks_per_tile):
    mesh = plsc.VectorSubcoreMesh(core_axis_name="c", subcore_axis_name="s")
    return pl.kernel(
        functools.partial(_sc_agg_body, chunks_per_tile),
        out_type=jax.ShapeDtypeStruct((NUM_CORES, N_PAD, H), jnp.float32),
        mesh=mesh,
        scratch_types=[
            pltpu.VMEM((CH,), jnp.int32),
            pltpu.VMEM((CH,), jnp.int32),
            pltpu.VMEM((CH, H), jnp.float32),
            pltpu.VMEM_SHARED((N_PAD, H), jnp.float32),
            pltpu.SemaphoreType.DMA,
        ],
    )(h, srcp, dstp, zeros)


def _sc_deg_body(chunks_per_tile, dst_hbm, out_hbm, didx_all, hist):
    cid = lax.axis_index("c")
    sid = lax.axis_index("s")
    wid = cid * NUM_SUBCORES + sid
    edges_per_tile = CH * chunks_per_tile

    def zbody(i, carry):
        hist[pl.ds(i * 16, 16)] = jnp.zeros((16,), jnp.float32)
        return carry

    lax.fori_loop(0, N_PAD // 16, zbody, 0)
    pltpu.sync_copy(dst_hbm.at[pl.ds(wid * edges_per_tile, edges_per_tile)],
                    didx_all)

    ones16 = jnp.ones((16,), jnp.float32)

    def body(g, carry):
        idx = didx_all[pl.ds(g * 16, 16)]
        plsc.addupdate_scatter(hist, [idx], ones16)
        return carry

    lax.fori_loop(0, edges_per_tile // 16, body, 0)
    pltpu.sync_copy(hist, out_hbm.at[wid])


def _sc_deg(dstp, chunks_per_tile):
    mesh = plsc.VectorSubcoreMesh(core_axis_name="c", subcore_axis_name="s")
    edges_per_tile = CH * chunks_per_tile
    return pl.kernel(
        functools.partial(_sc_deg_body, chunks_per_tile),
        out_type=jax.ShapeDtypeStruct((NUM_TILES, N_PAD), jnp.float32),
        mesh=mesh,
        scratch_types=[
            pltpu.VMEM((edges_per_tile,), jnp.int32),
            pltpu.VMEM((N_PAD,), jnp.float32),
        ],
        compiler_params=pltpu.CompilerParams(needs_layout_passes=False),
    )(dstp)


def _embed_body(x_ref, w_ref, b_ref, o_ref):
    o_ref[...] = jnp.maximum(
        jnp.dot(x_ref[...], w_ref[...], precision=HIGH) + b_ref[...], 0.0)


def _embed(x_pad, W_emb, b_emb):
    grid = (N_PAD // ROW_BLK,)
    return pl.pallas_call(
        _embed_body,
        grid=grid,
        in_specs=[
            pl.BlockSpec((ROW_BLK, H), lambda i: (i, 0)),
            pl.BlockSpec((H, H), lambda i: (0, 0)),
            pl.BlockSpec((1, H), lambda i: (0, 0)),
        ],
        out_specs=pl.BlockSpec((ROW_BLK, H), lambda i: (i, 0)),
        out_shape=jax.ShapeDtypeStruct((N_PAD, H), jnp.float32),
    )(x_pad, W_emb, b_emb.reshape(1, H))


def _recip_body(h_ref, o_ref):
    deg = jnp.sum(h_ref[...], axis=0)
    o_ref[...] = (1.0 / jnp.maximum(deg, 1.0))[:, None]


def _recip_deg(hists):
    return pl.pallas_call(
        _recip_body,
        out_shape=jax.ShapeDtypeStruct((N_PAD, 1), jnp.float32),
    )(hists)


def _layer_body(h_ref, a0_ref, a1_ref, rd_ref, ws_ref, wn_ref, b_ref, o_ref):
    agg = (a0_ref[...] + a1_ref[...]) * rd_ref[...]
    o_ref[...] = jnp.maximum(
        jnp.dot(h_ref[...], ws_ref[...], precision=HIGH)
        + jnp.dot(agg, wn_ref[...], precision=HIGH) + b_ref[...], 0.0)


def _layer(h, agg2, rd, Ws, Wn, b):
    grid = (N_PAD // ROW_BLK,)
    row_spec = pl.BlockSpec((ROW_BLK, H), lambda i: (i, 0))
    w_spec = pl.BlockSpec((H, H), lambda i: (0, 0))
    return pl.pallas_call(
        _layer_body,
        grid=grid,
        in_specs=[
            row_spec,
            row_spec,
            row_spec,
            pl.BlockSpec((ROW_BLK, 1), lambda i: (i, 0)),
            w_spec,
            w_spec,
            pl.BlockSpec((1, H), lambda i: (0, 0)),
        ],
        out_specs=row_spec,
        out_shape=jax.ShapeDtypeStruct((N_PAD, H), jnp.float32),
    )(h, agg2[0], agg2[1], rd, Ws, Wn, b.reshape(1, H))


def _pool_body(h_ref, bt_ref, w1_ref, b1_ref, w2_ref, b2_ref, o_ref,
               acc_ref, cnt_ref):
    i = pl.program_id(0)
    nsteps = pl.num_programs(0)

    @pl.when(i == 0)
    def _():
        acc_ref[...] = jnp.zeros_like(acc_ref)
        cnt_ref[...] = jnp.zeros_like(cnt_ref)

    b = bt_ref[0, 0, :]
    onehot = (b[:, None]
              == lax.broadcasted_iota(jnp.int32, (ROW_BLK, G), 1)
              ).astype(jnp.float32)
    acc_ref[...] += lax.dot_general(
        onehot, h_ref[...], (((0,), (0,)), ((), ())), precision=HIGH)
    cnt_ref[...] += lax.dot_general(
        onehot, jnp.ones((ROW_BLK, 1), jnp.float32),
        (((0,), (0,)), ((), ())), precision=HIGH)

    @pl.when(i == nsteps - 1)
    def _():
        pooled = acc_ref[...] / jnp.maximum(cnt_ref[...], 1.0)
        hid = jnp.maximum(
            jnp.dot(pooled, w1_ref[...], precision=HIGH) + b1_ref[...], 0.0)
        o_ref[...] = jnp.dot(hid, w2_ref[...], precision=HIGH) + b2_ref[...]


def _pool_head(h, batch3d, W1, b1, W2, b2):
    grid = (N_PAD // ROW_BLK,)
    return pl.pallas_call(
        _pool_body,
        grid=grid,
        in_specs=[
            pl.BlockSpec((ROW_BLK, H), lambda i: (i, 0)),
            pl.BlockSpec((1, 1, ROW_BLK), lambda i: (i, 0, 0)),
            pl.BlockSpec((H, H), lambda i: (0, 0)),
            pl.BlockSpec((1, H), lambda i: (0, 0)),
            pl.BlockSpec((H, 1), lambda i: (0, 0)),
            pl.BlockSpec((1, 1), lambda i: (0, 0)),
        ],
        out_specs=pl.BlockSpec((G, 1), lambda i: (0, 0)),
        out_shape=jax.ShapeDtypeStruct((G, 1), jnp.float32),
        scratch_shapes=[
            pltpu.VMEM((G, H), jnp.float32),
            pltpu.VMEM((G, 1), jnp.float32),
        ],
    )(h, batch3d, W1, b1.reshape(1, H), W2, b2.reshape(1, 1))


def kernel(x, edge_index, batch, W_emb, b_emb, W_self, W_nbr, b_conv,
           W1, b1, W2, b2):
    n = x.shape[0]
    e = edge_index.shape[1]
    num_layers = W_self.shape[0]

    chunks_per_tile = -(-e // (NUM_TILES * CH))
    e_pad = NUM_TILES * CH * chunks_per_tile

    src = edge_index[0]
    dst = edge_index[1]
    srcp = jnp.concatenate(
        [src, jnp.zeros((e_pad - e,), src.dtype)]).astype(jnp.int32)
    dstp = jnp.concatenate(
        [dst, jnp.full((e_pad - e,), N_PAD - 1, dst.dtype)]).astype(jnp.int32)

    x_pad = jnp.pad(x, ((0, N_PAD - n), (0, 0)))
    batch3d = jnp.pad(batch.astype(jnp.int32), (0, N_PAD - n),
                      constant_values=G).reshape(N_PAD // ROW_BLK, 1, ROW_BLK)
    zeros = jnp.zeros((N_PAD, H), jnp.float32)

    hists = _sc_deg(dstp, chunks_per_tile)
    rd = _recip_deg(hists)
    h = _embed(x_pad, W_emb, b_emb)
    for l in range(num_layers):
        agg2 = _sc_agg(h, srcp, dstp, zeros, chunks_per_tile)
        h = _layer(h, agg2, rd, W_self[l], W_nbr[l], b_conv[l])
    return _pool_head(h, batch3d, W1, b1, W2, b2)
